# Initial kernel scaffold; baseline (speedup 1.0000x reference)
import jax
import jax.numpy as jnp
from jax.experimental import pallas as pl


def kernel(logits, u, indices):
    raise NotImplementedError("write your pallas kernel here")


# speedup vs baseline: 15.4544x; 15.4544x over previous
import jax
import jax.numpy as jnp
from jax import lax
from jax.experimental import pallas as pl
from jax.experimental.pallas import tpu as pltpu
from jax.experimental.pallas import tpu_sc as plsc

NUM_LAYERS = 2
P = 512
NLEV = 16
OUT = 4096
SCALE = OUT // P
SLICING = 0.001

SC_CORES = 2
SC_SUBCORES = 16
SC_WORKERS = SC_CORES * SC_SUBCORES
ROWS_PER_TILE = P // SC_WORKERS
LANES = 16
CHUNKS = P // LANES


def _levels_body(logits_ref, u_ref, out_ref):
    lvl = lax.broadcasted_iota(jnp.int32, (P, NLEV), 1).astype(jnp.float32)
    for l in range(NUM_LAYERS):
        u = u_ref[l]
        g = -jnp.log(-jnp.log(u + 1e-20) + 1e-20)
        x = logits_ref[l] + g
        m = jnp.max(x, axis=1, keepdims=True)
        e = jnp.exp(x - m)
        s = jnp.sum(e, axis=1)
        w = jnp.sum(e * lvl, axis=1)
        out_ref[l, :] = w / s * SLICING


_levels_call = pl.pallas_call(
    _levels_body,
    out_shape=jax.ShapeDtypeStruct((NUM_LAYERS, P), jnp.float32),
)


def _gather_body(tab_hbm, idx_hbm, out_hbm, tab_v, idx_v, out_v):
    wid = lax.axis_index("s") * SC_CORES + lax.axis_index("c")
    rbase = wid * ROWS_PER_TILE
    pltpu.sync_copy(tab_hbm, tab_v)
    pltpu.sync_copy(idx_hbm.at[pl.ds(rbase, ROWS_PER_TILE)], idx_v)

    def row(j, carry):
        for l in range(NUM_LAYERS):
            for c in range(CHUNKS):
                iv = idx_v[j, pl.ds(c * LANES, LANES)]
                vals = plsc.load_gather(tab_v, [iv + l * P])
                out_v[l, j, pl.ds(c * LANES, LANES)] = vals
        return carry

    lax.fori_loop(0, ROWS_PER_TILE, row, 0)
    for l in range(NUM_LAYERS):
        pltpu.sync_copy(out_v.at[l], out_hbm.at[l, pl.ds(rbase, ROWS_PER_TILE)])


_gather_call_cache = []


def _gather_call(tab, indices):
    if not _gather_call_cache:
        _gather_call_cache.append(pl.kernel(
            _gather_body,
            out_type=jax.ShapeDtypeStruct((NUM_LAYERS, P, P), jnp.float32),
            mesh=plsc.VectorSubcoreMesh(core_axis_name="c", subcore_axis_name="s"),
            compiler_params=pltpu.CompilerParams(needs_layout_passes=False),
            scratch_types=[
                pltpu.VMEM((NUM_LAYERS * P,), jnp.float32),
                pltpu.VMEM((ROWS_PER_TILE, P), jnp.int32),
                pltpu.VMEM((NUM_LAYERS, ROWS_PER_TILE, P), jnp.float32),
            ],
        ))
    return _gather_call_cache[0](tab, indices)


ROWS_BLK = P // SCALE


def _upsample_body(g_ref, out_ref, r_ref):
    @pl.when(jnp.logical_and(pl.program_id(0) == 0, pl.program_id(1) == 0))
    def _():
        cols = lax.broadcasted_iota(jnp.int32, (P, OUT), 1)
        rows = lax.broadcasted_iota(jnp.int32, (P, OUT), 0)
        r_ref[...] = ((cols // SCALE) == rows).astype(jnp.bfloat16)

    g = g_ref[0].astype(jnp.bfloat16)
    w = jnp.dot(g, r_ref[...], preferred_element_type=jnp.float32)
    out_ref[0] = jnp.broadcast_to(w[:, None, :], (ROWS_BLK, SCALE, OUT)).reshape(
        P, OUT)


_upsample_call = pl.pallas_call(
    _upsample_body,
    grid=(NUM_LAYERS, OUT // P),
    in_specs=[pl.BlockSpec((1, ROWS_BLK, P), lambda l, k: (l, k, 0))],
    out_specs=pl.BlockSpec((1, P, OUT), lambda l, k: (l, k, 0)),
    out_shape=jax.ShapeDtypeStruct((NUM_LAYERS, OUT, OUT), jnp.float32),
    scratch_shapes=[pltpu.VMEM((P, OUT), jnp.bfloat16)],
)


def kernel(logits, u, indices):
    tab = _levels_call(logits, u).reshape(NUM_LAYERS * P)
    small = _gather_call(tab, indices)
    big = _upsample_call(small)
    return big[None]


# speedup vs baseline: 16.1060x; 1.0422x over previous
import jax
import jax.numpy as jnp
from jax import lax
from jax.experimental import pallas as pl
from jax.experimental.pallas import tpu as pltpu
from jax.experimental.pallas import tpu_sc as plsc

NUM_LAYERS = 2
P = 512
NLEV = 16
OUT = 4096
SCALE = OUT // P
SLICING = 0.001

SC_CORES = 2
SC_SUBCORES = 16
SC_WORKERS = SC_CORES * SC_SUBCORES
ROWS_PER_TILE = P // SC_WORKERS
LANES = 16
CHUNKS = P // LANES


def _levels_body(logits_ref, u_ref, out_ref):
    lvl = lax.broadcasted_iota(jnp.int32, (P, NLEV), 1).astype(jnp.float32)
    for l in range(NUM_LAYERS):
        u = u_ref[l]
        g = -jnp.log(-jnp.log(u + 1e-20) + 1e-20)
        x = logits_ref[l] + g
        m = jnp.max(x, axis=1, keepdims=True)
        e = jnp.exp(x - m)
        s = jnp.sum(e, axis=1)
        w = jnp.sum(e * lvl, axis=1)
        out_ref[l, :] = w / s * SLICING


_levels_call = pl.pallas_call(
    _levels_body,
    out_shape=jax.ShapeDtypeStruct((NUM_LAYERS, P), jnp.float32),
)


def _gather_body(tab_hbm, idx_hbm, out_hbm, tab_v, idx_v, out_v):
    wid = lax.axis_index("s") * SC_CORES + lax.axis_index("c")
    rbase = wid * ROWS_PER_TILE
    pltpu.sync_copy(tab_hbm, tab_v)
    pltpu.sync_copy(idx_hbm.at[pl.ds(rbase, ROWS_PER_TILE)], idx_v)

    @plsc.parallel_loop(0, ROWS_PER_TILE, unroll=2)
    def row(j):
        for c in range(CHUNKS):
            iv = idx_v[j, pl.ds(c * LANES, LANES)]
            for l in range(NUM_LAYERS):
                vals = plsc.load_gather(tab_v, [iv] if l == 0 else [iv + l * P])
                out_v[l, j, pl.ds(c * LANES, LANES)] = vals
    for l in range(NUM_LAYERS):
        pltpu.sync_copy(out_v.at[l], out_hbm.at[l, pl.ds(rbase, ROWS_PER_TILE)])


_gather_call_cache = []


def _gather_call(tab, indices):
    if not _gather_call_cache:
        _gather_call_cache.append(pl.kernel(
            _gather_body,
            out_type=jax.ShapeDtypeStruct((NUM_LAYERS, P, P), jnp.float32),
            mesh=plsc.VectorSubcoreMesh(core_axis_name="c", subcore_axis_name="s"),
            compiler_params=pltpu.CompilerParams(needs_layout_passes=False),
            scratch_types=[
                pltpu.VMEM((NUM_LAYERS * P,), jnp.float32),
                pltpu.VMEM((ROWS_PER_TILE, P), jnp.int32),
                pltpu.VMEM((NUM_LAYERS, ROWS_PER_TILE, P), jnp.float32),
            ],
        ))
    return _gather_call_cache[0](tab, indices)


ROWS_BLK = P // SCALE


def _upsample_body(g_ref, out_ref, r_ref):
    @pl.when(jnp.logical_and(pl.program_id(0) == 0, pl.program_id(1) == 0))
    def _():
        cols = lax.broadcasted_iota(jnp.int32, (P, OUT), 1)
        rows = lax.broadcasted_iota(jnp.int32, (P, OUT), 0)
        r_ref[...] = ((cols // SCALE) == rows).astype(jnp.bfloat16)

    g = g_ref[0].astype(jnp.bfloat16)
    w = jnp.dot(g, r_ref[...], preferred_element_type=jnp.float32)
    out_ref[0] = jnp.broadcast_to(w[:, None, :], (ROWS_BLK, SCALE, OUT)).reshape(
        P, OUT)


_upsample_call = pl.pallas_call(
    _upsample_body,
    grid=(NUM_LAYERS, OUT // P),
    in_specs=[pl.BlockSpec((1, ROWS_BLK, P), lambda l, k: (l, k, 0))],
    out_specs=pl.BlockSpec((1, P, OUT), lambda l, k: (l, k, 0)),
    out_shape=jax.ShapeDtypeStruct((NUM_LAYERS, OUT, OUT), jnp.float32),
    scratch_shapes=[pltpu.VMEM((P, OUT), jnp.bfloat16)],
)


def kernel(logits, u, indices):
    tab = _levels_call(logits, u).reshape(NUM_LAYERS * P)
    small = _gather_call(tab, indices)
    big = _upsample_call(small)
    return big[None]
